# baseline (speedup 1.0000x reference)
import functools

import jax
import jax.numpy as jnp
from jax import lax
from jax.experimental import pallas as pl
from jax.experimental.pallas import tpu as pltpu
from jax.experimental.pallas import tpu_sc as plsc

_E = 64
_CH = 128
_NBUF = 5

_info = plsc.get_sparse_core_info()
_NC, _NS = _info.num_cores, _info.num_subcores
_NW = _NC * _NS


def _make_sc(nplanes):
    n = nplanes * 1024
    pw = n // _NW
    ng = pw // _CH
    assert pw % _CH == 0 and ng % _NBUF == 0

    def body(idx_hbm, table_hbm, out_hbm, idx_v, rows_v,
             g0, g1, g2, g3, g4, wsem):
        gsems = (g0, g1, g2, g3, g4)
        wid = lax.axis_index("s") * _NC + lax.axis_index("c")
        base = wid * pw
        pltpu.sync_copy(idx_hbm.at[pl.ds(base, pw)], idx_v)

        for s in range(_NBUF):
            pltpu.async_copy(
                table_hbm.at[idx_v.at[pl.ds(_CH * s, _CH)]], rows_v.at[s],
                gsems[s])

        def step(g, carry):
            for s in range(_NBUF):
                i = g * _NBUF + s
                pltpu.make_async_copy(
                    table_hbm.at[idx_v.at[pl.ds(i * _CH, _CH)]],
                    rows_v.at[s], gsems[s]).wait()
                pltpu.async_copy(
                    rows_v.at[s], out_hbm.at[pl.ds(base + i * _CH, _CH)],
                    wsem).wait()
                nxt = i + _NBUF

                @pl.when(nxt < ng)
                def _():
                    pltpu.async_copy(
                        table_hbm.at[idx_v.at[pl.ds(nxt * _CH, _CH)]],
                        rows_v.at[s], gsems[s])
            return carry

        lax.fori_loop(0, ng // _NBUF, step, 0)

    return pl.kernel(
        body,
        mesh=plsc.VectorSubcoreMesh(core_axis_name="c",
                                    subcore_axis_name="s"),
        out_type=jax.ShapeDtypeStruct((n, _E), jnp.float32),
        scratch_types=[
            pltpu.VMEM((pw,), jnp.int32),
            pltpu.VMEM((_NBUF, _CH, _E), jnp.float32),
            pltpu.SemaphoreType.DMA,
            pltpu.SemaphoreType.DMA,
            pltpu.SemaphoreType.DMA,
            pltpu.SemaphoreType.DMA,
            pltpu.SemaphoreType.DMA,
            pltpu.SemaphoreType.DMA,
        ],
        compiler_params=pltpu.CompilerParams(use_tc_tiling_on_sc=False),
    )


_NP_A = 60
_NP_B = 40
_sc_a = _make_sc(_NP_A)
_sc_b = _make_sc(_NP_B)

_RP = 10


def _tc_body_first(in_ref, out_ref):
    for i in range(_RP):
        sub = in_ref[pl.ds(i * 512, 512), :]
        t0 = jnp.transpose(sub[:, 0:_E])
        t1 = jnp.transpose(sub[:, _E:2 * _E])
        out_ref[:, i, :, 0:512] = jnp.broadcast_to(t0[None], (4, _E, 512))
        out_ref[:, i, :, 512:1024] = jnp.broadcast_to(t1[None], (4, _E, 512))


def _tc_body_second(in_ref, prev_ref, out_ref):
    _tc_body_first(in_ref, out_ref)


_OUT_SHAPE = jax.ShapeDtypeStruct((4, 100, _E, 1024), jnp.float32)

_tc_a = pl.pallas_call(
    _tc_body_first,
    grid=(_NP_A // _RP,),
    in_specs=[pl.BlockSpec((_RP * 512, 128), lambda g: (g, 0))],
    out_specs=pl.BlockSpec((4, _RP, _E, 1024), lambda g: (0, g, 0, 0)),
    out_shape=_OUT_SHAPE,
)

_tc_b = pl.pallas_call(
    _tc_body_second,
    grid=(_NP_B // _RP,),
    in_specs=[
        pl.BlockSpec((_RP * 512, 128), lambda g: (g, 0)),
        pl.BlockSpec(memory_space=pl.ANY),
    ],
    out_specs=pl.BlockSpec((4, _RP, _E, 1024),
                           lambda g: (0, g + _NP_A // _RP, 0, 0)),
    out_shape=_OUT_SHAPE,
    input_output_aliases={1: 0},
)


def kernel(x, table):
    xt = x.transpose(1, 2, 0).reshape(100, 1024).astype(jnp.int32)
    xt = xt.reshape(100, 2, 512).transpose(0, 2, 1).reshape(-1)
    inter_a = _sc_a(xt[:_NP_A * 1024], table)
    inter_b = _sc_b(xt[_NP_A * 1024:], table)
    out4 = _tc_a(inter_a.reshape(_NP_A * 512, 128))
    out4 = _tc_b(inter_b.reshape(_NP_B * 512, 128), out4)
    return (out4.reshape(80, 5, _E, 1024).transpose(3, 0, 1, 2))


# speedup vs baseline: 1.0839x; 1.0839x over previous
import jax
import jax.numpy as jnp
from jax import lax
from jax.experimental import pallas as pl
from jax.experimental.pallas import tpu as pltpu
from jax.experimental.pallas import tpu_sc as plsc

_N = 102400
_E = 64
_CH = 128
_NBUF = 5

_info = plsc.get_sparse_core_info()
_NC, _NS = _info.num_cores, _info.num_subcores
_NW = _NC * _NS
_PW = _N // _NW
_NG = _PW // _CH


def _sc_body(idx_hbm, table_hbm, out_hbm, idx_v, rows_v,
             g0, g1, g2, g3, g4, wsem):
    gsems = (g0, g1, g2, g3, g4)
    wid = lax.axis_index("s") * _NC + lax.axis_index("c")
    base = wid * _PW
    pltpu.sync_copy(idx_hbm.at[pl.ds(base, _PW)], idx_v)

    for s in range(_NBUF):
        pltpu.async_copy(
            table_hbm.at[idx_v.at[pl.ds(_CH * s, _CH)]], rows_v.at[s],
            gsems[s])

    def step(g, carry):
        for s in range(_NBUF):
            i = g * _NBUF + s
            pltpu.make_async_copy(
                table_hbm.at[idx_v.at[pl.ds(i * _CH, _CH)]], rows_v.at[s],
                gsems[s]).wait()
            pltpu.async_copy(
                rows_v.at[s], out_hbm.at[pl.ds(base + i * _CH, _CH)],
                wsem).wait()
            nxt = i + _NBUF

            @pl.when(nxt < _NG)
            def _():
                pltpu.async_copy(
                    table_hbm.at[idx_v.at[pl.ds(nxt * _CH, _CH)]],
                    rows_v.at[s], gsems[s])
        return carry

    lax.fori_loop(0, _NG // _NBUF, step, 0)


_sc = pl.kernel(
    _sc_body,
    mesh=plsc.VectorSubcoreMesh(core_axis_name="c", subcore_axis_name="s"),
    out_type=jax.ShapeDtypeStruct((_N, _E), jnp.float32),
    scratch_types=[
        pltpu.VMEM((_PW,), jnp.int32),
        pltpu.VMEM((_NBUF, _CH, _E), jnp.float32),
        pltpu.SemaphoreType.DMA,
        pltpu.SemaphoreType.DMA,
        pltpu.SemaphoreType.DMA,
        pltpu.SemaphoreType.DMA,
        pltpu.SemaphoreType.DMA,
        pltpu.SemaphoreType.DMA,
    ],
    compiler_params=pltpu.CompilerParams(use_tc_tiling_on_sc=False),
)


_RP = 10


def _tc_body(in_ref, out_ref):
    for i in range(_RP):
        sub = in_ref[pl.ds(i * 512, 512), :]
        t0 = jnp.transpose(sub[:, 0:_E])
        t1 = jnp.transpose(sub[:, _E:2 * _E])
        out_ref[:, i, :, 0:512] = jnp.broadcast_to(t0[None], (4, _E, 512))
        out_ref[:, i, :, 512:1024] = jnp.broadcast_to(t1[None], (4, _E, 512))


_tc = pl.pallas_call(
    _tc_body,
    grid=(100 // _RP,),
    in_specs=[pl.BlockSpec((_RP * 512, 128), lambda g: (g, 0))],
    out_specs=pl.BlockSpec((4, _RP, _E, 1024), lambda g: (0, g, 0, 0)),
    out_shape=jax.ShapeDtypeStruct((4, 100, _E, 1024), jnp.float32),
)


def kernel(x, table):
    xt = x.transpose(1, 2, 0).reshape(100, 1024).astype(jnp.int32)
    xt = xt.reshape(100, 2, 512).transpose(0, 2, 1).reshape(-1)
    inter = _sc(xt, table)
    out4 = _tc(inter.reshape(51200, 128))
    return (out4.reshape(80, 5, _E, 1024).transpose(3, 0, 1, 2))
